# baseline (speedup 1.0000x reference)
import jax
import jax.numpy as jnp
from jax.experimental import pallas as pl
from jax.experimental.pallas import tpu as pltpu

_TAU = 3.0
_DECAY = 1.0 - 1.0 / _TAU
_VTH = 1.2
_B, _L, _T = 1024, 14, 64
_B_BLK = 512

_W_NAMES = (
    'b1_bridge', 'b1_inside', 'b2_bridge', 'b2_inside', 'b3_bridge',
    'b3_inside', 'b4_bridge', 'b4_inside', 'b5_bridge', 'b5_inside',
    'b6_bridge', 'b6_inside', 'r21', 'r32', 'r43', 'r54', 'r65',
)


def _mm(s, w):
    return jnp.dot(s, w, preferred_element_type=jnp.float32)


def _lif_nospike(v, x):
    v = v * _DECAY + x
    return jnp.where(v >= _VTH, 0.0, v)


def _lif_spike(v, x, iota_f):
    v = v * _DECAY + x
    fire = v >= _VTH
    idx = jnp.argmax(v, axis=1, keepdims=True)
    spike = jnp.where((iota_f == idx) & fire, 1.0, 0.0).astype(jnp.bfloat16)
    v = jnp.where(fire, 0.0, v)
    return v, spike


def _exchange(n):
    r = jax.lax.broadcasted_iota(jnp.int32, (n, n), 0)
    c = jax.lax.broadcasted_iota(jnp.int32, (n, n), 1)
    return jnp.where(r + c == n - 1, 1.0, 0.0)


def _fwd(x_ref, b1b_r, b1i_r, b2b_r, b2i_r, b3b_r, b3i_r, b4b_r, b4i_r,
         b5b_r, b5i_r, b6b_r, b6i_r, r21_r, r32_r, r43_r, r54_r, r65_r,
         o2, o3, o4, o5, o6):
    bsz = x_ref.shape[2]
    iota64 = jax.lax.broadcasted_iota(jnp.int32, (bsz, 64), 1)
    iota128 = jax.lax.broadcasted_iota(jnp.int32, (bsz, 128), 1)

    j64 = _exchange(64)
    j128 = _exchange(128)

    def flip(w, jl, jr):
        if jl is None:
            f = _mm(w, jr)
        else:
            f = _mm(jl, _mm(w, jr))
        return f.astype(jnp.bfloat16)

    b1b = flip(b1b_r[...], None, j64)
    b1i = flip(b1i_r[...], j64, j64)
    b2b = flip(b2b_r[...], j64, j128)
    b2i = flip(b2i_r[...], j128, j128)
    b3b = flip(b3b_r[...], j128, j128)
    b3i = flip(b3i_r[...], j128, j128)
    b4b = flip(b4b_r[...], j128, j128)
    b4i = flip(b4i_r[...], j128, j128)
    b5b = flip(b5b_r[...], j128, j128)
    b5i = flip(b5i_r[...], j128, j128)
    b6b = flip(b6b_r[...], j128, j128)
    b6i = flip(b6i_r[...], j128, j128)
    r21 = flip(r21_r[...], j128, j64)
    r32 = flip(r32_r[...], j128, j128)
    r43 = flip(r43_r[...], j128, j128)
    r54 = flip(r54_r[...], j128, j128)
    r65 = flip(r65_r[...], j128, j128)

    def step(t, carry):
        v1, v2, v3, v4, v5, v6 = carry
        return _one_step(t, v1, v2, v3, v4, v5, v6)

    def _one_step(t, v1, v2, v3, v4, v5, v6):
        x_t = x_ref[t]
        z1 = jax.lax.dot_general(
            x_t, b1b, (((0,), (0,)), ((), ())),
            preferred_element_type=jnp.float32)
        v1, s = _lif_spike(v1, z1, iota64)
        v1 = _lif_nospike(v1, _mm(s, b1i))
        v2, s = _lif_spike(v2, _mm(s, b2b), iota128)
        v2 = _lif_nospike(v2, _mm(s, b2i))
        v3, s = _lif_spike(v3, _mm(s, b3b), iota128)
        v3 = _lif_nospike(v3, _mm(s, b3i))
        v4, s = _lif_spike(v4, _mm(s, b4b), iota128)
        v4 = _lif_nospike(v4, _mm(s, b4i))
        v5, s = _lif_spike(v5, _mm(s, b5b), iota128)
        v5 = _lif_nospike(v5, _mm(s, b5i))
        v6, s = _lif_spike(v6, _mm(s, b6b), iota128)
        v6 = _lif_nospike(v6, _mm(s, b6i))
        v5, s = _lif_spike(v5, _mm(s, r65), iota128)
        v5 = _lif_nospike(v5, _mm(s, b5i))
        v4, s = _lif_spike(v4, _mm(s, r54), iota128)
        v4 = _lif_nospike(v4, _mm(s, b4i))
        v3, s = _lif_spike(v3, _mm(s, r43), iota128)
        v3 = _lif_nospike(v3, _mm(s, b3i))
        v2, s = _lif_spike(v2, _mm(s, r32), iota128)
        v2 = _lif_nospike(v2, _mm(s, b2i))
        v1, s = _lif_spike(v1, _mm(s, r21), iota64)
        v1 = _lif_nospike(v1, _mm(s, b1i))
        return (v1, v2, v3, v4, v5, v6)

    init = (
        jnp.zeros((bsz, 64), jnp.float32),
        jnp.zeros((bsz, 128), jnp.float32),
        jnp.zeros((bsz, 128), jnp.float32),
        jnp.zeros((bsz, 128), jnp.float32),
        jnp.zeros((bsz, 128), jnp.float32),
        jnp.zeros((bsz, 128), jnp.float32),
    )
    _UNROLL = 4

    def steps(i, carry):
        t0 = i * _UNROLL
        for k in range(_UNROLL):
            carry = step(t0 + k, carry)
        return carry

    _, v2, v3, v4, v5, v6 = jax.lax.fori_loop(0, _T // _UNROLL, steps, init)
    def unflip(v):
        return jnp.dot(v, j128, preferred_element_type=jnp.float32,
                       precision=jax.lax.Precision.HIGHEST)

    o2[...] = jnp.exp(unflip(v2))
    o3[...] = jnp.exp(unflip(v3))
    o4[...] = jnp.exp(unflip(v4))
    o5[...] = jnp.exp(unflip(v5))
    o6[...] = jnp.exp(unflip(v6))


def kernel(x, params):
    ws = [params[n] for n in _W_NAMES]
    xs = jnp.transpose(x, (2, 1, 0)).astype(jnp.bfloat16)
    nblk = _B // _B_BLK
    in_specs = [pl.BlockSpec((_T, _L, _B_BLK), lambda i: (0, 0, i))]
    in_specs += [pl.BlockSpec(w.shape, lambda i: (0, 0)) for w in ws]
    out_specs = [pl.BlockSpec((_B_BLK, 128), lambda i: (i, 0))] * 5
    out_shape = [jax.ShapeDtypeStruct((_B, 128), jnp.float32)] * 5
    outs = pl.pallas_call(
        _fwd,
        grid=(nblk,),
        in_specs=in_specs,
        out_specs=out_specs,
        out_shape=out_shape,
        compiler_params=pltpu.CompilerParams(
            dimension_semantics=("parallel",),
            vmem_limit_bytes=48 * 1024 * 1024,
        ),
    )(xs, *ws)
    return tuple(outs)


# speedup vs baseline: 1.0353x; 1.0353x over previous
import jax
import jax.numpy as jnp
from jax.experimental import pallas as pl
from jax.experimental.pallas import tpu as pltpu

_TAU = 3.0
_DECAY = 1.0 - 1.0 / _TAU
_VTH = 1.2
_B, _L, _T = 1024, 14, 64
_B_BLK = 512

_W_NAMES = (
    'b1_bridge', 'b1_inside', 'b2_bridge', 'b2_inside', 'b3_bridge',
    'b3_inside', 'b4_bridge', 'b4_inside', 'b5_bridge', 'b5_inside',
    'b6_bridge', 'b6_inside', 'r21', 'r32', 'r43', 'r54', 'r65',
)


def _mm(s, w):
    return jnp.dot(s, w, preferred_element_type=jnp.float32)


def _lif_nospike(v, x):
    v = v * _DECAY + x
    return jnp.where(v >= _VTH, 0.0, v)


def _lif_spike(v, x, iota_f):
    v = v * _DECAY + x
    fire = v >= _VTH
    idx = jnp.argmax(v, axis=1, keepdims=True)
    spike = jnp.where((iota_f == idx) & fire, 1.0, 0.0)
    v = jnp.where(fire, 0.0, v)
    return v, spike


def _exchange(n):
    r = jax.lax.broadcasted_iota(jnp.int32, (n, n), 0)
    c = jax.lax.broadcasted_iota(jnp.int32, (n, n), 1)
    return jnp.where(r + c == n - 1, 1.0, 0.0)


def _fwd(x_ref, b1b_r, b1i_r, b2b_r, b2i_r, b3b_r, b3i_r, b4b_r, b4i_r,
         b5b_r, b5i_r, b6b_r, b6i_r, r21_r, r32_r, r43_r, r54_r, r65_r,
         o2, o3, o4, o5, o6):
    bsz = x_ref.shape[2]
    iota64 = jax.lax.broadcasted_iota(jnp.int32, (bsz, 64), 1)
    iota128 = jax.lax.broadcasted_iota(jnp.int32, (bsz, 128), 1)

    j64 = _exchange(64)
    j128 = _exchange(128)

    def flip(w, jl, jr):
        if jl is None:
            return _mm(w, jr)
        return _mm(jl, _mm(w, jr))

    b1b = flip(b1b_r[...], None, j64)
    b1i = flip(b1i_r[...], j64, j64)
    b2b = flip(b2b_r[...], j64, j128)
    b2i = flip(b2i_r[...], j128, j128)
    b3b = flip(b3b_r[...], j128, j128)
    b3i = flip(b3i_r[...], j128, j128)
    b4b = flip(b4b_r[...], j128, j128)
    b4i = flip(b4i_r[...], j128, j128)
    b5b = flip(b5b_r[...], j128, j128)
    b5i = flip(b5i_r[...], j128, j128)
    b6b = flip(b6b_r[...], j128, j128)
    b6i = flip(b6i_r[...], j128, j128)
    r21 = flip(r21_r[...], j128, j64)
    r32 = flip(r32_r[...], j128, j128)
    r43 = flip(r43_r[...], j128, j128)
    r54 = flip(r54_r[...], j128, j128)
    r65 = flip(r65_r[...], j128, j128)

    def step(t, carry):
        v1, v2, v3, v4, v5, v6 = carry
        return _one_step(t, v1, v2, v3, v4, v5, v6)

    def _one_step(t, v1, v2, v3, v4, v5, v6):
        x_t = x_ref[t]
        z1 = jax.lax.dot_general(
            x_t, b1b, (((0,), (0,)), ((), ())),
            preferred_element_type=jnp.float32)
        v1, s = _lif_spike(v1, z1, iota64)
        v1 = _lif_nospike(v1, _mm(s, b1i))
        v2, s = _lif_spike(v2, _mm(s, b2b), iota128)
        v2 = _lif_nospike(v2, _mm(s, b2i))
        v3, s = _lif_spike(v3, _mm(s, b3b), iota128)
        v3 = _lif_nospike(v3, _mm(s, b3i))
        v4, s = _lif_spike(v4, _mm(s, b4b), iota128)
        v4 = _lif_nospike(v4, _mm(s, b4i))
        v5, s = _lif_spike(v5, _mm(s, b5b), iota128)
        v5 = _lif_nospike(v5, _mm(s, b5i))
        v6, s = _lif_spike(v6, _mm(s, b6b), iota128)
        v6 = _lif_nospike(v6, _mm(s, b6i))
        v5, s = _lif_spike(v5, _mm(s, r65), iota128)
        v5 = _lif_nospike(v5, _mm(s, b5i))
        v4, s = _lif_spike(v4, _mm(s, r54), iota128)
        v4 = _lif_nospike(v4, _mm(s, b4i))
        v3, s = _lif_spike(v3, _mm(s, r43), iota128)
        v3 = _lif_nospike(v3, _mm(s, b3i))
        v2, s = _lif_spike(v2, _mm(s, r32), iota128)
        v2 = _lif_nospike(v2, _mm(s, b2i))
        v1, s = _lif_spike(v1, _mm(s, r21), iota64)
        v1 = _lif_nospike(v1, _mm(s, b1i))
        return (v1, v2, v3, v4, v5, v6)

    init = (
        jnp.zeros((bsz, 64), jnp.float32),
        jnp.zeros((bsz, 128), jnp.float32),
        jnp.zeros((bsz, 128), jnp.float32),
        jnp.zeros((bsz, 128), jnp.float32),
        jnp.zeros((bsz, 128), jnp.float32),
        jnp.zeros((bsz, 128), jnp.float32),
    )
    _UNROLL = 4

    def steps(i, carry):
        t0 = i * _UNROLL
        for k in range(_UNROLL):
            carry = step(t0 + k, carry)
        return carry

    _, v2, v3, v4, v5, v6 = jax.lax.fori_loop(0, _T // _UNROLL, steps, init)
    def unflip(v):
        return jnp.dot(v, j128, preferred_element_type=jnp.float32,
                       precision=jax.lax.Precision.HIGHEST)

    o2[...] = jnp.exp(unflip(v2))
    o3[...] = jnp.exp(unflip(v3))
    o4[...] = jnp.exp(unflip(v4))
    o5[...] = jnp.exp(unflip(v5))
    o6[...] = jnp.exp(unflip(v6))


def kernel(x, params):
    ws = [params[n] for n in _W_NAMES]
    xs = jnp.transpose(x, (2, 1, 0))
    nblk = _B // _B_BLK
    in_specs = [pl.BlockSpec((_T, _L, _B_BLK), lambda i: (0, 0, i))]
    in_specs += [pl.BlockSpec(w.shape, lambda i: (0, 0)) for w in ws]
    out_specs = [pl.BlockSpec((_B_BLK, 128), lambda i: (i, 0))] * 5
    out_shape = [jax.ShapeDtypeStruct((_B, 128), jnp.float32)] * 5
    outs = pl.pallas_call(
        _fwd,
        grid=(nblk,),
        in_specs=in_specs,
        out_specs=out_specs,
        out_shape=out_shape,
        compiler_params=pltpu.CompilerParams(
            dimension_semantics=("parallel",),
            vmem_limit_bytes=48 * 1024 * 1024,
        ),
    )(xs, *ws)
    return tuple(outs)


# speedup vs baseline: 1.0753x; 1.0386x over previous
import jax
import jax.numpy as jnp
from jax.experimental import pallas as pl
from jax.experimental.pallas import tpu as pltpu

_TAU = 3.0
_DECAY = 1.0 - 1.0 / _TAU
_VTH = 1.2
_B, _L, _T = 1024, 14, 64
_B_BLK = 512

_W_NAMES = (
    'b1_bridge', 'b1_inside', 'b2_bridge', 'b2_inside', 'b3_bridge',
    'b3_inside', 'b4_bridge', 'b4_inside', 'b5_bridge', 'b5_inside',
    'b6_bridge', 'b6_inside', 'r21', 'r32', 'r43', 'r54', 'r65',
)


def _mm(s, w):
    return jnp.dot(s, w, preferred_element_type=jnp.float32)


def _lif_nospike(v, x):
    v = v * _DECAY + x
    return jnp.where(v >= _VTH, 0.0, v)


def _lif_spike(v, x, iota_f):
    v = v * _DECAY + x
    fire = v >= _VTH
    idx = jnp.argmax(v, axis=1, keepdims=True)
    spike = jnp.where((iota_f == idx) & fire, 1.0, 0.0)
    v = jnp.where(fire, 0.0, v)
    return v, spike


def _exchange(n):
    r = jax.lax.broadcasted_iota(jnp.int32, (n, n), 0)
    c = jax.lax.broadcasted_iota(jnp.int32, (n, n), 1)
    return jnp.where(r + c == n - 1, 1.0, 0.0)


def _fwd(x_ref, b1b_r, b1i_r, b2b_r, b2i_r, b3b_r, b3i_r, b4b_r, b4i_r,
         b5b_r, b5i_r, b6b_r, b6i_r, r21_r, r32_r, r43_r, r54_r, r65_r,
         o2, o3, o4, o5, o6):
    bsz = x_ref.shape[2]
    iota64 = jax.lax.broadcasted_iota(jnp.int32, (bsz, 64), 1)
    iota128 = jax.lax.broadcasted_iota(jnp.int32, (bsz, 128), 1)

    j64 = _exchange(64)
    j128 = _exchange(128)

    def flip(w, jl, jr):
        if jl is None:
            return _mm(w, jr)
        return _mm(jl, _mm(w, jr))

    b1b = flip(b1b_r[...], None, j64)
    b1i = flip(b1i_r[...], j64, j64)
    b2b = flip(b2b_r[...], j64, j128)
    b2i = flip(b2i_r[...], j128, j128)
    b3b = flip(b3b_r[...], j128, j128)
    b3i = flip(b3i_r[...], j128, j128)
    b4b = flip(b4b_r[...], j128, j128)
    b4i = flip(b4i_r[...], j128, j128)
    b5b = flip(b5b_r[...], j128, j128)
    b5i = flip(b5i_r[...], j128, j128)
    b6b = flip(b6b_r[...], j128, j128)
    b6i = flip(b6i_r[...], j128, j128)
    r21 = flip(r21_r[...], j128, j64)
    r32 = flip(r32_r[...], j128, j128)
    r43 = flip(r43_r[...], j128, j128)
    r54 = flip(r54_r[...], j128, j128)
    r65 = flip(r65_r[...], j128, j128)

    def step(t, carry):
        v1, v2, v3, v4, v5, v6 = carry
        return _one_step(t, v1, v2, v3, v4, v5, v6)

    def _one_step(t, v1, v2, v3, v4, v5, v6):
        x_t = x_ref[t]
        z1 = jax.lax.dot_general(
            x_t, b1b, (((0,), (0,)), ((), ())),
            preferred_element_type=jnp.float32)
        v1, s = _lif_spike(v1, z1, iota64)
        v1 = _lif_nospike(v1, _mm(s, b1i))
        v2, s = _lif_spike(v2, _mm(s, b2b), iota128)
        v2 = _lif_nospike(v2, _mm(s, b2i))
        v3, s = _lif_spike(v3, _mm(s, b3b), iota128)
        v3 = _lif_nospike(v3, _mm(s, b3i))
        v4, s = _lif_spike(v4, _mm(s, b4b), iota128)
        v4 = _lif_nospike(v4, _mm(s, b4i))
        v5, s = _lif_spike(v5, _mm(s, b5b), iota128)
        v5 = _lif_nospike(v5, _mm(s, b5i))
        v6, s = _lif_spike(v6, _mm(s, b6b), iota128)
        v6 = _lif_nospike(v6, _mm(s, b6i))
        v5, s = _lif_spike(v5, _mm(s, r65), iota128)
        v5 = _lif_nospike(v5, _mm(s, b5i))
        v4, s = _lif_spike(v4, _mm(s, r54), iota128)
        v4 = _lif_nospike(v4, _mm(s, b4i))
        v3, s = _lif_spike(v3, _mm(s, r43), iota128)
        v3 = _lif_nospike(v3, _mm(s, b3i))
        v2, s = _lif_spike(v2, _mm(s, r32), iota128)
        v2 = _lif_nospike(v2, _mm(s, b2i))
        v1, s = _lif_spike(v1, _mm(s, r21), iota64)
        v1 = _lif_nospike(v1, _mm(s, b1i))
        return (v1, v2, v3, v4, v5, v6)

    init = (
        jnp.zeros((bsz, 64), jnp.float32),
        jnp.zeros((bsz, 128), jnp.float32),
        jnp.zeros((bsz, 128), jnp.float32),
        jnp.zeros((bsz, 128), jnp.float32),
        jnp.zeros((bsz, 128), jnp.float32),
        jnp.zeros((bsz, 128), jnp.float32),
    )
    _UNROLL = 8

    def steps(i, carry):
        t0 = i * _UNROLL
        for k in range(_UNROLL):
            carry = step(t0 + k, carry)
        return carry

    _, v2, v3, v4, v5, v6 = jax.lax.fori_loop(0, _T // _UNROLL, steps, init)
    def unflip(v):
        return jnp.dot(v, j128, preferred_element_type=jnp.float32,
                       precision=jax.lax.Precision.HIGHEST)

    o2[...] = jnp.exp(unflip(v2))
    o3[...] = jnp.exp(unflip(v3))
    o4[...] = jnp.exp(unflip(v4))
    o5[...] = jnp.exp(unflip(v5))
    o6[...] = jnp.exp(unflip(v6))


def kernel(x, params):
    ws = [params[n] for n in _W_NAMES]
    xs = jnp.transpose(x, (2, 1, 0))
    nblk = _B // _B_BLK
    in_specs = [pl.BlockSpec((_T, _L, _B_BLK), lambda i: (0, 0, i))]
    in_specs += [pl.BlockSpec(w.shape, lambda i: (0, 0)) for w in ws]
    out_specs = [pl.BlockSpec((_B_BLK, 128), lambda i: (i, 0))] * 5
    out_shape = [jax.ShapeDtypeStruct((_B, 128), jnp.float32)] * 5
    outs = pl.pallas_call(
        _fwd,
        grid=(nblk,),
        in_specs=in_specs,
        out_specs=out_specs,
        out_shape=out_shape,
        compiler_params=pltpu.CompilerParams(
            dimension_semantics=("parallel",),
            vmem_limit_bytes=48 * 1024 * 1024,
        ),
    )(xs, *ws)
    return tuple(outs)
